# Initial kernel scaffold; baseline (speedup 1.0000x reference)
import jax
import jax.numpy as jnp
from jax.experimental import pallas as pl


def kernel(x, S0, W_k, W_v, W_q, W_erase, W_write):
    raise NotImplementedError("write your pallas kernel here")


# speedup vs baseline: 2.9426x; 2.9426x over previous
import functools

import jax
import jax.numpy as jnp
from jax.experimental import pallas as pl
from jax.experimental.pallas import tpu as pltpu

TC = 128
BB = 32


def _ntm_kernel(x_ref, s0_ref, w_ref, out_ref, sfin_ref,
                s_scr, k_scr, q_scr, e_scr, wv_scr, *, n_chunks, n):
    ic = pl.program_id(1)

    @pl.when(ic == 0)
    def _init():
        s_scr[...] = s0_ref[...]

    d = x_ref.shape[-1]
    xb = x_ref[...].reshape(TC * BB, d)
    p = jnp.dot(xb, w_ref[...], preferred_element_type=jnp.float32)

    k = jax.nn.softmax(p[:, 0 * n:1 * n], axis=-1)
    v = p[:, 1 * n:2 * n]
    q = jax.nn.softmax(p[:, 2 * n:3 * n], axis=-1)
    e = jax.nn.sigmoid(p[:, 3 * n:4 * n])
    w = jax.nn.sigmoid(p[:, 4 * n:5 * n])

    k_scr[...] = k.reshape(TC, BB, n)
    q_scr[...] = q.reshape(TC, BB, n)
    e_scr[...] = e.reshape(TC, BB, n)
    wv_scr[...] = (w * v).reshape(TC, BB, n)

    def step(s, carry):
        k_s = k_scr[s]
        q_s = q_scr[s]
        e_s = e_scr[s]
        wv_s = wv_scr[s]
        kt = k_s[:, None, :]
        s_new = s_scr[...] * (1.0 - e_s[:, :, None] * kt) \
            + wv_s[:, :, None] * kt
        s_scr[...] = s_new
        out_ref[s] = jnp.sum(s_new * q_s[:, None, :], axis=2)
        return carry

    jax.lax.fori_loop(0, TC, step, 0)
    out_ref[...] = jnp.tanh(out_ref[...])

    @pl.when(ic == n_chunks - 1)
    def _fin():
        sfin_ref[...] = s_scr[...]


def kernel(x, S0, W_k, W_v, W_q, W_erase, W_write):
    T, B, D = x.shape
    N = W_k.shape[0]
    n_chunks = T // TC
    nb = B // BB

    w_all = jnp.concatenate([W_k, W_v, W_q, W_erase, W_write], axis=0).T

    kern = functools.partial(_ntm_kernel, n_chunks=n_chunks, n=N)

    out, s_fin = pl.pallas_call(
        kern,
        grid=(nb, n_chunks),
        in_specs=[
            pl.BlockSpec((TC, BB, D), lambda ib, ic: (ic, ib, 0)),
            pl.BlockSpec((BB, N, N), lambda ib, ic: (ib, 0, 0)),
            pl.BlockSpec((D, 5 * N), lambda ib, ic: (0, 0)),
        ],
        out_specs=[
            pl.BlockSpec((TC, BB, N), lambda ib, ic: (ic, ib, 0)),
            pl.BlockSpec((BB, N, N), lambda ib, ic: (ib, 0, 0)),
        ],
        out_shape=[
            jax.ShapeDtypeStruct((T, B, N), jnp.float32),
            jax.ShapeDtypeStruct((B, N, N), jnp.float32),
        ],
        scratch_shapes=[
            pltpu.VMEM((BB, N, N), jnp.float32),
            pltpu.VMEM((TC, BB, N), jnp.float32),
            pltpu.VMEM((TC, BB, N), jnp.float32),
            pltpu.VMEM((TC, BB, N), jnp.float32),
            pltpu.VMEM((TC, BB, N), jnp.float32),
        ],
        compiler_params=pltpu.CompilerParams(
            dimension_semantics=("parallel", "arbitrary"),
            vmem_limit_bytes=56 * 1024 * 1024,
        ),
        name="ntm_cell",
    )(x, S0, w_all)
    return out, s_fin
